# baseline (speedup 1.0000x reference)
import functools

import jax
import jax.numpy as jnp
from jax import lax
from jax.experimental import pallas as pl
from jax.experimental.pallas import tpu as pltpu
from jax.experimental.pallas import tpu_sc as plsc

_ROWS = 8192
_D = 1024
_NC = 2
_NS = 16
_NW = _NC * _NS
_RPW = _ROWS // _NW
_CHUNK = 16
_NCHUNK = _RPW // _CHUNK // 2
_NBUF = 3

_mesh = plsc.VectorSubcoreMesh(core_axis_name="c", subcore_axis_name="s")


class _Chain:
    def __init__(self, table, out, bufs, sins, souts, row0):
        self.table, self.out = table, out
        self.bufs, self.sins, self.souts = bufs, sins, souts
        self.row0 = row0
        self.inc = [None] * _NBUF
        self.outc = [None] * _NBUF

    def prime(self):
        for i in range(min(_NBUF, _NCHUNK)):
            self.inc[i] = pltpu.async_copy(
                self.table.at[pl.ds(self.row0 + i * _CHUNK, _CHUNK)],
                self.bufs[i], self.sins[i])

    def step(self, k):
        b = k % _NBUF
        self.inc[b].wait()
        self.outc[b] = pltpu.async_copy(
            self.bufs[b],
            self.out.at[pl.ds(self.row0 + k * _CHUNK, _CHUNK)],
            self.souts[b])
        j = k + _NBUF
        if j < _NCHUNK:
            self.outc[b].wait()
            self.outc[b] = None
            self.inc[b] = pltpu.async_copy(
                self.table.at[pl.ds(self.row0 + j * _CHUNK, _CHUNK)],
                self.bufs[b], self.sins[b])

    def drain(self):
        for b in range(_NBUF):
            if self.outc[b] is not None:
                self.outc[b].wait()


@functools.partial(
    pl.kernel,
    out_type=jax.ShapeDtypeStruct((_ROWS, _D), jnp.float32),
    mesh=_mesh,
    scratch_types=[
        pltpu.VMEM((_NBUF, _CHUNK, _D), jnp.float32),
        pltpu.VMEM_SHARED((_NS, _NBUF, _CHUNK, _D), jnp.float32),
    ] + [pltpu.SemaphoreType.DMA] * (4 * _NBUF),
)
def _pe_copy(table_hbm, out_hbm, tbuf, sbuf, *sems):
    cid = lax.axis_index("c")
    sid = lax.axis_index("s")
    wid = sid * _NC + cid
    base = wid * _RPW
    half = _NCHUNK * _CHUNK
    chain_t = _Chain(
        table_hbm, out_hbm,
        [tbuf.at[b] for b in range(_NBUF)],
        sems[0:_NBUF], sems[_NBUF:2 * _NBUF], base)
    chain_s = _Chain(
        table_hbm, out_hbm,
        [sbuf.at[sid, b] for b in range(_NBUF)],
        sems[2 * _NBUF:3 * _NBUF], sems[3 * _NBUF:4 * _NBUF], base + half)
    chain_t.prime()
    chain_s.prime()
    for k in range(_NCHUNK):
        chain_t.step(k)
        chain_s.step(k)
    chain_t.drain()
    chain_s.drain()


def kernel(x, pe_weight):
    del x
    return _pe_copy(pe_weight)[None]


# speedup vs baseline: 1.0028x; 1.0028x over previous
import functools

import jax
import jax.numpy as jnp
from jax import lax
from jax.experimental import pallas as pl
from jax.experimental.pallas import tpu as pltpu
from jax.experimental.pallas import tpu_sc as plsc

_ROWS = 8192
_D = 1024
_NC = 2
_NS = 16
_NW = _NC * _NS
_RPW = _ROWS // _NW
_CHUNK = 16
_NCHUNK = _RPW // _CHUNK
_NBUF = 6
_LAG = 3

_mesh = plsc.VectorSubcoreMesh(core_axis_name="c", subcore_axis_name="s")


@functools.partial(
    pl.kernel,
    out_type=jax.ShapeDtypeStruct((_ROWS, _D), jnp.float32),
    mesh=_mesh,
    scratch_types=[
        pltpu.VMEM((_NBUF, _CHUNK, _D), jnp.float32),
    ] + [pltpu.SemaphoreType.DMA] * (2 * _NBUF),
)
def _pe_copy(table_hbm, out_hbm, buf, *sems):
    sins = sems[:_NBUF]
    souts = sems[_NBUF:]
    wid = lax.axis_index("s") * _NC + lax.axis_index("c")
    base = wid * _RPW
    in_copies = [None] * _NBUF
    out_copies = [None] * _NBUF

    for i in range(_NBUF - _LAG):
        in_copies[i] = pltpu.async_copy(
            table_hbm.at[pl.ds(base + i * _CHUNK, _CHUNK)],
            buf.at[i], sins[i])
    for i in range(_NCHUNK):
        b = i % _NBUF
        in_copies[b].wait()
        out_copies[b] = pltpu.async_copy(
            buf.at[b], out_hbm.at[pl.ds(base + i * _CHUNK, _CHUNK)], souts[b])
        j = i + _NBUF - _LAG
        if j < _NCHUNK:
            jb = j % _NBUF
            if out_copies[jb] is not None:
                out_copies[jb].wait()
                out_copies[jb] = None
            in_copies[jb] = pltpu.async_copy(
                table_hbm.at[pl.ds(base + j * _CHUNK, _CHUNK)],
                buf.at[jb], sins[jb])
    for b in range(_NBUF):
        if out_copies[b] is not None:
            out_copies[b].wait()


def kernel(x, pe_weight):
    del x
    return _pe_copy(pe_weight)[None]
